# baseline (speedup 1.0000x reference)
import jax
import jax.numpy as jnp
from jax.experimental import pallas as pl
from jax.experimental.pallas import tpu as pltpu

_NPOINT = 512


def _fps_kernel(x_ref, xt_ref, nrow_ref, ncol_ref, out_ref, d_ref):
    n = x_ref.shape[1]
    corr = jnp.dot(x_ref[0], xt_ref[0], preferred_element_type=jnp.float32)
    d_ref[...] = (ncol_ref[0] + nrow_ref[0]) - 2.0 * corr

    iota = jax.lax.broadcasted_iota(jnp.int32, (1, n), 1)

    def step(k, carry):
        dists, f = carry
        row = d_ref[pl.ds(f, 1), :]
        dists = jnp.minimum(dists, row)
        out_ref[0, pl.ds(k, 1), :] = jnp.full((1, 1), f, jnp.int32)
        m = jnp.max(dists)
        nf = jnp.min(jnp.where(dists == m, iota, n)).astype(jnp.int32)
        return dists, nf

    dists0 = jnp.full((1, n), 1e10, dtype=jnp.float32)
    jax.lax.fori_loop(0, _NPOINT, step, (dists0, jnp.int32(0)))


def kernel(points_xyz, features):
    b, n, _ = points_xyz.shape
    feats_t = jnp.transpose(features, (0, 2, 1))
    x = jnp.concatenate([points_xyz, feats_t], axis=2)
    c = x.shape[2]
    norms = jnp.sum(x ** 2, axis=-1)
    xt = jnp.transpose(x, (0, 2, 1))
    nrow = norms[:, None, :]
    ncol = norms[:, :, None]

    idxs = pl.pallas_call(
        _fps_kernel,
        grid=(b,),
        in_specs=[
            pl.BlockSpec((1, n, c), lambda i: (i, 0, 0)),
            pl.BlockSpec((1, c, n), lambda i: (i, 0, 0)),
            pl.BlockSpec((1, 1, n), lambda i: (i, 0, 0)),
            pl.BlockSpec((1, n, 1), lambda i: (i, 0, 0)),
        ],
        out_specs=pl.BlockSpec((1, _NPOINT, 1), lambda i: (i, 0, 0)),
        out_shape=jax.ShapeDtypeStruct((b, _NPOINT, 1), jnp.int32),
        scratch_shapes=[pltpu.VMEM((n, n), jnp.float32)],
        compiler_params=pltpu.CompilerParams(
            dimension_semantics=("arbitrary",),
        ),
    )(x, xt, nrow, ncol)
    return idxs[..., 0]


# speedup vs baseline: 1.9462x; 1.9462x over previous
import jax
import jax.numpy as jnp
from jax.experimental import pallas as pl
from jax.experimental.pallas import tpu as pltpu

_NPOINT = 512


def _fps_kernel(x_ref, xt_ref, nrow_ref, ncol_ref, out_ref, d_ref):
    n = x_ref.shape[1]
    corr = jnp.dot(x_ref[0], xt_ref[0], preferred_element_type=jnp.float32)
    d_ref[...] = (ncol_ref[0] + nrow_ref[0]) - 2.0 * corr

    nsub = n // 128
    rowiota = jax.lax.broadcasted_iota(jnp.int32, (nsub, 1), 0)

    def step(k, carry):
        dists, f = carry
        row = d_ref[pl.ds(f, 1), :].reshape(nsub, 128)
        dists = jnp.minimum(dists, row)
        out_ref[0, pl.ds(k, 1), :] = jnp.full((1, 1), f, jnp.int32)
        mv = jnp.max(dists, axis=1, keepdims=True)
        mi = jnp.argmax(dists, axis=1, keepdims=True).astype(jnp.int32)
        m = jnp.max(mv)
        nf = jnp.min(
            jnp.where(mv == m, rowiota * 128 + mi, jnp.int32(1 << 30))
        ).astype(jnp.int32)
        return dists, nf

    dists0 = jnp.full((nsub, 128), 1e10, dtype=jnp.float32)
    jax.lax.fori_loop(0, _NPOINT, step, (dists0, jnp.int32(0)))


def kernel(points_xyz, features):
    b, n, _ = points_xyz.shape
    feats_t = jnp.transpose(features, (0, 2, 1))
    x = jnp.concatenate([points_xyz, feats_t], axis=2)
    c = x.shape[2]
    norms = jnp.sum(x ** 2, axis=-1)
    xt = jnp.transpose(x, (0, 2, 1))
    nrow = norms[:, None, :]
    ncol = norms[:, :, None]

    idxs = pl.pallas_call(
        _fps_kernel,
        grid=(b,),
        in_specs=[
            pl.BlockSpec((1, n, c), lambda i: (i, 0, 0)),
            pl.BlockSpec((1, c, n), lambda i: (i, 0, 0)),
            pl.BlockSpec((1, 1, n), lambda i: (i, 0, 0)),
            pl.BlockSpec((1, n, 1), lambda i: (i, 0, 0)),
        ],
        out_specs=pl.BlockSpec((1, _NPOINT, 1), lambda i: (i, 0, 0)),
        out_shape=jax.ShapeDtypeStruct((b, _NPOINT, 1), jnp.int32),
        scratch_shapes=[pltpu.VMEM((n, n), jnp.float32)],
        compiler_params=pltpu.CompilerParams(
            dimension_semantics=("parallel",),
        ),
    )(x, xt, nrow, ncol)
    return idxs[..., 0]


# speedup vs baseline: 2.9579x; 1.5198x over previous
import jax
import jax.numpy as jnp
from jax.experimental import pallas as pl
from jax.experimental.pallas import tpu as pltpu

_NPOINT = 512
_NB = 2


def _fps_kernel(x_ref, xt_ref, nrow_ref, ncol_ref, out_ref, d_ref):
    n = x_ref.shape[1]
    for b in range(_NB):
        corr = jnp.dot(x_ref[b], xt_ref[b], preferred_element_type=jnp.float32)
        d_ref[b] = (ncol_ref[b] + nrow_ref[b]) - 2.0 * corr

    nsub = n // 128
    rowiota = jax.lax.broadcasted_iota(jnp.int32, (nsub, 1), 0)

    def step(k, carry):
        new = []
        for b in range(_NB):
            dists, f = carry[2 * b], carry[2 * b + 1]
            row = d_ref[b, pl.ds(f, 1), :].reshape(nsub, 128)
            dists = jnp.minimum(dists, row)
            out_ref[b, pl.ds(k, 1), :] = jnp.full((1, 1), f, jnp.int32)
            mv = jnp.max(dists, axis=1, keepdims=True)
            mi = jnp.argmax(dists, axis=1, keepdims=True).astype(jnp.int32)
            m = jnp.max(mv)
            nf = jnp.min(
                jnp.where(mv == m, rowiota * 128 + mi, jnp.int32(1 << 30))
            ).astype(jnp.int32)
            new += [dists, nf]
        return tuple(new)

    dists0 = jnp.full((nsub, 128), 1e10, dtype=jnp.float32)
    init = []
    for b in range(_NB):
        init += [dists0, jnp.int32(0)]
    jax.lax.fori_loop(0, _NPOINT, step, tuple(init))


def kernel(points_xyz, features):
    b, n, _ = points_xyz.shape
    feats_t = jnp.transpose(features, (0, 2, 1))
    x = jnp.concatenate([points_xyz, feats_t], axis=2)
    c = x.shape[2]
    norms = jnp.sum(x ** 2, axis=-1)
    xt = jnp.transpose(x, (0, 2, 1))
    nrow = norms[:, None, :]
    ncol = norms[:, :, None]

    idxs = pl.pallas_call(
        _fps_kernel,
        grid=(b // _NB,),
        in_specs=[
            pl.BlockSpec((_NB, n, c), lambda i: (i, 0, 0)),
            pl.BlockSpec((_NB, c, n), lambda i: (i, 0, 0)),
            pl.BlockSpec((_NB, 1, n), lambda i: (i, 0, 0)),
            pl.BlockSpec((_NB, n, 1), lambda i: (i, 0, 0)),
        ],
        out_specs=pl.BlockSpec((_NB, _NPOINT, 1), lambda i: (i, 0, 0)),
        out_shape=jax.ShapeDtypeStruct((b, _NPOINT, 1), jnp.int32),
        scratch_shapes=[pltpu.VMEM((_NB, n, n), jnp.float32)],
        compiler_params=pltpu.CompilerParams(
            dimension_semantics=("parallel",),
        ),
    )(x, xt, nrow, ncol)
    return idxs[..., 0]


# speedup vs baseline: 4.6472x; 1.5711x over previous
import jax
import jax.numpy as jnp
from jax.experimental import pallas as pl
from jax.experimental.pallas import tpu as pltpu

_NPOINT = 512
_NB = 2

_DIMNUMS = (((0,), (0,)), ((), ()))


def _fps_kernel(xt_ref, nrow_ref, out_ref, d_ref):
    n = xt_ref.shape[2]
    for b in range(_NB):
        corr = jax.lax.dot_general(
            xt_ref[b], xt_ref[b], _DIMNUMS, preferred_element_type=jnp.float32
        )
        ncol = jnp.transpose(nrow_ref[b], (1, 0))
        d_ref[b] = (ncol + nrow_ref[b]) - 2.0 * corr

    nsub = n // 128
    rowiota = jax.lax.broadcasted_iota(jnp.int32, (nsub, 1), 0)

    def argmax_combine(mv, mi):
        while mv.shape[0] > 1:
            h = mv.shape[0] // 2
            lo_v, hi_v = mv[:h], mv[h:]
            lo_i, hi_i = mi[:h], mi[h:]
            take_hi = hi_v > lo_v
            mv = jnp.where(take_hi, hi_v, lo_v)
            mi = jnp.where(take_hi, hi_i, lo_i)
        return jnp.max(mi)

    def step(k, carry):
        new = []
        for b in range(_NB):
            dists, f = carry[2 * b], carry[2 * b + 1]
            row = d_ref[b, pl.ds(f, 1), :].reshape(nsub, 128)
            dists = jnp.minimum(dists, row)
            out_ref[b, pl.ds(k, 1), :] = jnp.full((1, 1), f, jnp.int32)
            mv = jnp.max(dists, axis=1, keepdims=True)
            mi = jnp.argmax(dists, axis=1, keepdims=True).astype(jnp.int32)
            nf = argmax_combine(mv, rowiota * 128 + mi).astype(jnp.int32)
            new += [dists, nf]
        return tuple(new)

    dists0 = jnp.full((nsub, 128), 1e10, dtype=jnp.float32)
    init = []
    for b in range(_NB):
        init += [dists0, jnp.int32(0)]
    jax.lax.fori_loop(0, _NPOINT, step, tuple(init))


def kernel(points_xyz, features):
    b, n, _ = points_xyz.shape
    feats_t = jnp.transpose(features, (0, 2, 1))
    x = jnp.concatenate([points_xyz, feats_t], axis=2)
    c = x.shape[2]
    norms = jnp.sum(x ** 2, axis=-1)
    nrow = norms[:, None, :]
    xt = jnp.concatenate(
        [jnp.transpose(points_xyz, (0, 2, 1)), features], axis=1
    )

    bp = -(-b // _NB) * _NB
    if bp != b:
        pad = [(0, bp - b)] + [(0, 0)] * 2
        xt = jnp.pad(xt, pad)
        nrow = jnp.pad(nrow, pad)

    idxs = pl.pallas_call(
        _fps_kernel,
        grid=(bp // _NB,),
        in_specs=[
            pl.BlockSpec((_NB, c, n), lambda i: (i, 0, 0)),
            pl.BlockSpec((_NB, 1, n), lambda i: (i, 0, 0)),
        ],
        out_specs=pl.BlockSpec((_NB, _NPOINT, 1), lambda i: (i, 0, 0)),
        out_shape=jax.ShapeDtypeStruct((bp, _NPOINT, 1), jnp.int32),
        scratch_shapes=[pltpu.VMEM((_NB, n, n), jnp.float32)],
        compiler_params=pltpu.CompilerParams(
            dimension_semantics=("parallel",),
        ),
    )(xt, nrow)
    return idxs[:b, :, 0]
